# baseline (speedup 1.0000x reference)
import functools

import jax
import jax.numpy as jnp
from jax import lax
from jax.experimental import pallas as pl
from jax.experimental.pallas import tpu as pltpu
from jax.experimental.pallas import tpu_sc as plsc

BATCH = 16384
CUST_VOCAB = 264055
CUST_DIM = 18
PROD_DIM = 7
PROD_VOCAB = 129
PROD_OH = 136
TBL_W = 128
DENSE_DIM = 13
H0, H1 = 1024, 512

NC, NS = 2, 16
NW = NC * NS
BPW = BATCH // NW
IDX_CHUNK = 128


_TP_COLS = 4096
_TP_GRID = (CUST_VOCAB + _TP_COLS - 1) // _TP_COLS


def _tpad_body(wt_ref, out_ref):
    eye = jnp.eye(CUST_DIM, TBL_W, dtype=jnp.float32)
    out_ref[...] = lax.dot_general(
        wt_ref[...], eye, (((0,), (0,)), ((), ())),
        preferred_element_type=jnp.float32)


def _tc_transpose_pad(wt):
    return pl.pallas_call(
        _tpad_body,
        grid=(_TP_GRID,),
        in_specs=[pl.BlockSpec((CUST_DIM, _TP_COLS), lambda i: (0, i))],
        out_specs=pl.BlockSpec((_TP_COLS, TBL_W), lambda i: (i, 0)),
        out_shape=jax.ShapeDtypeStruct((CUST_VOCAB, TBL_W), jnp.float32),
        compiler_params=pltpu.CompilerParams(
            dimension_semantics=("arbitrary",),
        ),
    )(wt)


@functools.lru_cache(maxsize=4)
def _make_sc_gather(batch):
    mesh = plsc.VectorSubcoreMesh(core_axis_name="c", subcore_axis_name="s")
    bpw = batch // NW

    @functools.partial(
        pl.kernel,
        mesh=mesh,
        out_type=jax.ShapeDtypeStruct((batch, TBL_W), jnp.float32),
        scratch_types=[
            pltpu.VMEM((bpw,), jnp.int32),
            pltpu.VMEM((bpw, TBL_W), jnp.float32),
            pltpu.SemaphoreType.DMA,
        ],
        compiler_params=pltpu.CompilerParams(use_tc_tiling_on_sc=False),
    )
    def _sc_gather(cid_hbm, wc_hbm, cust_out, cidx_v, crow_v, sem):
        wid = lax.axis_index("s") * NC + lax.axis_index("c")
        base = wid * bpw
        pltpu.sync_copy(cid_hbm.at[pl.ds(base, bpw)], cidx_v)
        nchunks = bpw // IDX_CHUNK
        for j in range(nchunks):
            sl = pl.ds(j * IDX_CHUNK, IDX_CHUNK)
            pltpu.async_copy(wc_hbm.at[cidx_v.at[sl]], crow_v.at[sl], sem)
        for j in range(nchunks):
            sl = pl.ds(j * IDX_CHUNK, IDX_CHUNK)
            pltpu.make_async_copy(wc_hbm.at[cidx_v.at[sl]], crow_v.at[sl],
                                  sem).wait()
        pltpu.sync_copy(crow_v, cust_out.at[pl.ds(base, bpw)])

    return _sc_gather


_ROWS = 2048
_NB = BATCH // _ROWS


def _mlp_body(cust_ref, pidx_ref, denset_ref, w1a_ref, p2_ref, w1c_ref,
              b1_ref, w2_ref, b2_ref, w3_ref, b3_ref, out_ref):
    h = jnp.dot(cust_ref[...], w1a_ref[...],
                preferred_element_type=jnp.float32)
    pidx = pidx_ref[0, 0, :]
    oh = (lax.broadcasted_iota(jnp.int32, (_ROWS, PROD_OH), 1)
          == pidx[:, None]).astype(jnp.float32)
    h = h + jnp.dot(oh, p2_ref[...], preferred_element_type=jnp.float32)
    h = h + lax.dot_general(denset_ref[...], w1c_ref[...],
                            (((0,), (0,)), ((), ())),
                            preferred_element_type=jnp.float32)
    h = jnp.maximum(h + b1_ref[...], 0.0)
    h = jnp.dot(h.astype(jnp.bfloat16), w2_ref[...],
                preferred_element_type=jnp.float32) + b2_ref[...]
    h = jnp.maximum(h, 0.0)
    o = jnp.dot(h, w3_ref[...], preferred_element_type=jnp.float32) + b3_ref[...]
    out_ref[...] = 1.0 / (1.0 + jnp.exp(-o))


def _tc_mlp(cust_emb, pidx3, denset, w1a, p2, w1c, b1, w2, b2, w3, b3,
            blk_off, nblk):
    full = lambda shape: pl.BlockSpec(shape, lambda i: (0, 0))
    rows = nblk * _ROWS
    return pl.pallas_call(
        _mlp_body,
        grid=(nblk,),
        in_specs=[
            pl.BlockSpec((_ROWS, TBL_W), lambda i: (i, 0)),
            pl.BlockSpec((1, 1, _ROWS), lambda i: (i + blk_off, 0, 0)),
            pl.BlockSpec((DENSE_DIM, _ROWS), lambda i: (0, i + blk_off)),
            full((TBL_W, H0)),
            full((PROD_OH, H0)),
            full((DENSE_DIM, H0)),
            full((1, H0)),
            full((H0, H1)),
            full((1, H1)),
            full((H1, 1)),
            full((1, 1)),
        ],
        out_specs=pl.BlockSpec((_ROWS, 1), lambda i: (i, 0)),
        out_shape=jax.ShapeDtypeStruct((rows, 1), jnp.float32),
        compiler_params=pltpu.CompilerParams(
            dimension_semantics=("arbitrary",),
        ),
    )(cust_emb, pidx3, denset, w1a, p2, w1c, b1, w2, b2, w3, b3)


_NSPLIT = 2
_BATCH_H = BATCH // _NSPLIT
_NB_H = _BATCH_H // _ROWS


def kernel(core_cust_id_input, prod_code_input, dense_input, W_cust, W_prod,
           W1, b1, W2, b2, W3, b3):
    wc_pad = _tc_transpose_pad(W_cust.T)
    w1a = jnp.pad(W1[:CUST_DIM], ((0, TBL_W - CUST_DIM), (0, 0)))
    p2 = jnp.pad(W_prod @ W1[CUST_DIM:CUST_DIM + PROD_DIM],
                 ((0, PROD_OH - PROD_VOCAB), (0, 0)))
    w1c = W1[CUST_DIM + PROD_DIM:]
    pidx3 = prod_code_input.reshape(_NB, 1, _ROWS)
    denset = dense_input.T
    w2b = W2.astype(jnp.bfloat16)
    gather = _make_sc_gather(_BATCH_H)
    outs = []
    for s in range(_NSPLIT):
        cid = lax.slice_in_dim(core_cust_id_input, s * _BATCH_H,
                               (s + 1) * _BATCH_H)
        ce = gather(cid, wc_pad)
        outs.append(_tc_mlp(ce, pidx3, denset,
                            w1a, p2, w1c, b1.reshape(1, H0),
                            w2b, b2.reshape(1, H1), W3, b3.reshape(1, 1),
                            s * _NB_H, _NB_H))
    return jnp.concatenate(outs, axis=0)


# speedup vs baseline: 1.0292x; 1.0292x over previous
import functools

import jax
import jax.numpy as jnp
from jax import lax
from jax.experimental import pallas as pl
from jax.experimental.pallas import tpu as pltpu
from jax.experimental.pallas import tpu_sc as plsc

BATCH = 16384
CUST_VOCAB = 264055
CUST_DIM = 18
PROD_DIM = 7
PROD_VOCAB = 129
PROD_OH = 136
TBL_W = 128
DENSE_DIM = 13
H0, H1 = 1024, 512

NC, NS = 2, 16
NW = NC * NS
BPW = BATCH // NW
IDX_CHUNK = 128


_TP_COLS = 4096
_TP_GRID = (CUST_VOCAB + _TP_COLS - 1) // _TP_COLS


def _tpad_body(wt_ref, out_ref):
    eye = jnp.eye(CUST_DIM, TBL_W, dtype=jnp.float32)
    out_ref[...] = lax.dot_general(
        wt_ref[...], eye, (((0,), (0,)), ((), ())),
        preferred_element_type=jnp.float32)


def _tc_transpose_pad(wt):
    return pl.pallas_call(
        _tpad_body,
        grid=(_TP_GRID,),
        in_specs=[pl.BlockSpec((CUST_DIM, _TP_COLS), lambda i: (0, i))],
        out_specs=pl.BlockSpec((_TP_COLS, TBL_W), lambda i: (i, 0)),
        out_shape=jax.ShapeDtypeStruct((CUST_VOCAB, TBL_W), jnp.float32),
        compiler_params=pltpu.CompilerParams(
            dimension_semantics=("arbitrary",),
        ),
    )(wt)


@functools.lru_cache(maxsize=4)
def _make_sc_gather(batch):
    mesh = plsc.VectorSubcoreMesh(core_axis_name="c", subcore_axis_name="s")
    bpw = batch // NW

    @functools.partial(
        pl.kernel,
        mesh=mesh,
        out_type=jax.ShapeDtypeStruct((batch, TBL_W), jnp.float32),
        scratch_types=[
            pltpu.VMEM((bpw,), jnp.int32),
            pltpu.VMEM((bpw, TBL_W), jnp.float32),
            pltpu.SemaphoreType.DMA,
        ],
        compiler_params=pltpu.CompilerParams(use_tc_tiling_on_sc=False),
    )
    def _sc_gather(cid_hbm, wc_hbm, cust_out, cidx_v, crow_v, sem):
        wid = lax.axis_index("s") * NC + lax.axis_index("c")
        base = wid * bpw
        pltpu.sync_copy(cid_hbm.at[pl.ds(base, bpw)], cidx_v)
        nchunks = bpw // IDX_CHUNK
        for j in range(nchunks):
            sl = pl.ds(j * IDX_CHUNK, IDX_CHUNK)
            pltpu.async_copy(wc_hbm.at[cidx_v.at[sl]], crow_v.at[sl], sem)
        for j in range(nchunks):
            sl = pl.ds(j * IDX_CHUNK, IDX_CHUNK)
            pltpu.make_async_copy(wc_hbm.at[cidx_v.at[sl]], crow_v.at[sl],
                                  sem).wait()
        pltpu.sync_copy(crow_v, cust_out.at[pl.ds(base, bpw)])

    return _sc_gather


_ROWS = 2048
_NB = BATCH // _ROWS


def _mlp_body(cust_ref, pidx_ref, denset_ref, w1a_ref, p2_ref, w1c_ref,
              b1_ref, w2_ref, b2_ref, w3_ref, b3_ref, out_ref):
    h = jnp.dot(cust_ref[...], w1a_ref[...],
                preferred_element_type=jnp.float32)
    pidx = pidx_ref[0, 0, :]
    oh = (lax.broadcasted_iota(jnp.int32, (_ROWS, PROD_OH), 1)
          == pidx[:, None]).astype(jnp.float32)
    h = h + jnp.dot(oh, p2_ref[...], preferred_element_type=jnp.float32)
    h = h + lax.dot_general(denset_ref[...], w1c_ref[...],
                            (((0,), (0,)), ((), ())),
                            preferred_element_type=jnp.float32)
    h = jnp.maximum(h + b1_ref[...], 0.0)
    h = jnp.dot(h.astype(jnp.bfloat16), w2_ref[...],
                preferred_element_type=jnp.float32) + b2_ref[...]
    h = jnp.maximum(h, 0.0)
    o = jnp.dot(h, w3_ref[...], preferred_element_type=jnp.float32) + b3_ref[...]
    out_ref[...] = 1.0 / (1.0 + jnp.exp(-o))


def _tc_mlp(cust_emb, pidx3, denset, w1a, p2, w1c, b1, w2, b2, w3, b3,
            blk_off, nblk):
    full = lambda shape: pl.BlockSpec(shape, lambda i: (0, 0))
    rows = nblk * _ROWS
    return pl.pallas_call(
        _mlp_body,
        grid=(nblk,),
        in_specs=[
            pl.BlockSpec((_ROWS, TBL_W), lambda i: (i, 0)),
            pl.BlockSpec((1, 1, _ROWS), lambda i: (i + blk_off, 0, 0)),
            pl.BlockSpec((DENSE_DIM, _ROWS), lambda i: (0, i + blk_off)),
            full((TBL_W, H0)),
            full((PROD_OH, H0)),
            full((DENSE_DIM, H0)),
            full((1, H0)),
            full((H0, H1)),
            full((1, H1)),
            full((H1, 1)),
            full((1, 1)),
        ],
        out_specs=pl.BlockSpec((_ROWS, 1), lambda i: (i, 0)),
        out_shape=jax.ShapeDtypeStruct((rows, 1), jnp.float32),
        compiler_params=pltpu.CompilerParams(
            dimension_semantics=("arbitrary",),
        ),
    )(cust_emb, pidx3, denset, w1a, p2, w1c, b1, w2, b2, w3, b3)


_NSPLIT = 1
_BATCH_H = BATCH // _NSPLIT
_NB_H = _BATCH_H // _ROWS


def kernel(core_cust_id_input, prod_code_input, dense_input, W_cust, W_prod,
           W1, b1, W2, b2, W3, b3):
    wc_pad = _tc_transpose_pad(W_cust.T)
    w1a = jnp.pad(W1[:CUST_DIM], ((0, TBL_W - CUST_DIM), (0, 0)))
    p2 = jnp.pad(W_prod @ W1[CUST_DIM:CUST_DIM + PROD_DIM],
                 ((0, PROD_OH - PROD_VOCAB), (0, 0)))
    w1c = W1[CUST_DIM + PROD_DIM:]
    pidx3 = prod_code_input.reshape(_NB, 1, _ROWS)
    denset = dense_input.T
    w2b = W2.astype(jnp.bfloat16)
    gather = _make_sc_gather(_BATCH_H)
    outs = []
    for s in range(_NSPLIT):
        cid = lax.slice_in_dim(core_cust_id_input, s * _BATCH_H,
                               (s + 1) * _BATCH_H)
        ce = gather(cid, wc_pad)
        outs.append(_tc_mlp(ce, pidx3, denset,
                            w1a, p2, w1c, b1.reshape(1, H0),
                            w2b, b2.reshape(1, H1), W3, b3.reshape(1, 1),
                            s * _NB_H, _NB_H))
    return jnp.concatenate(outs, axis=0)


# speedup vs baseline: 1.0292x; 1.0000x over previous
import functools

import jax
import jax.numpy as jnp
from jax import lax
from jax.experimental import pallas as pl
from jax.experimental.pallas import tpu as pltpu
from jax.experimental.pallas import tpu_sc as plsc

BATCH = 16384
CUST_VOCAB = 264055
CUST_DIM = 18
PROD_DIM = 7
PROD_VOCAB = 129
PROD_OH = 136
TBL_W = 128
DENSE_DIM = 13
H0, H1 = 1024, 512

NC, NS = 2, 16
NW = NC * NS
BPW = BATCH // NW
IDX_CHUNK = 128


_TP_COLS = 4096
_TP_GRID = (CUST_VOCAB + _TP_COLS - 1) // _TP_COLS


def _tpad_body(wt_ref, out_ref):
    eye = jnp.eye(CUST_DIM, CUST_DIM, dtype=jnp.float32)
    out_ref[:, :CUST_DIM] = lax.dot_general(
        wt_ref[...], eye, (((0,), (0,)), ((), ())),
        preferred_element_type=jnp.float32)


def _tc_transpose_pad(wt):
    return pl.pallas_call(
        _tpad_body,
        grid=(_TP_GRID,),
        in_specs=[pl.BlockSpec((CUST_DIM, _TP_COLS), lambda i: (0, i))],
        out_specs=pl.BlockSpec((_TP_COLS, TBL_W), lambda i: (i, 0)),
        out_shape=jax.ShapeDtypeStruct((CUST_VOCAB, TBL_W), jnp.float32),
        compiler_params=pltpu.CompilerParams(
            dimension_semantics=("arbitrary",),
        ),
    )(wt)


@functools.lru_cache(maxsize=4)
def _make_sc_gather(batch):
    mesh = plsc.VectorSubcoreMesh(core_axis_name="c", subcore_axis_name="s")
    bpw = batch // NW

    @functools.partial(
        pl.kernel,
        mesh=mesh,
        out_type=jax.ShapeDtypeStruct((batch, TBL_W), jnp.float32),
        scratch_types=[
            pltpu.VMEM((bpw,), jnp.int32),
            pltpu.VMEM((bpw, TBL_W), jnp.float32),
            pltpu.SemaphoreType.DMA,
        ],
        compiler_params=pltpu.CompilerParams(use_tc_tiling_on_sc=False),
    )
    def _sc_gather(cid_hbm, wc_hbm, cust_out, cidx_v, crow_v, sem):
        wid = lax.axis_index("s") * NC + lax.axis_index("c")
        base = wid * bpw
        pltpu.sync_copy(cid_hbm.at[pl.ds(base, bpw)], cidx_v)
        nchunks = bpw // IDX_CHUNK
        for j in range(nchunks):
            sl = pl.ds(j * IDX_CHUNK, IDX_CHUNK)
            pltpu.async_copy(wc_hbm.at[cidx_v.at[sl]], crow_v.at[sl], sem)
        for j in range(nchunks):
            sl = pl.ds(j * IDX_CHUNK, IDX_CHUNK)
            pltpu.make_async_copy(wc_hbm.at[cidx_v.at[sl]], crow_v.at[sl],
                                  sem).wait()
        pltpu.sync_copy(crow_v, cust_out.at[pl.ds(base, bpw)])

    return _sc_gather


_ROWS = 2048
_NB = BATCH // _ROWS


def _mlp_body(cust_ref, pidx_ref, denset_ref, w1a_ref, p2_ref, w1c_ref,
              b1_ref, w2_ref, b2_ref, w3_ref, b3_ref, out_ref):
    lane = lax.broadcasted_iota(jnp.int32, (_ROWS, TBL_W), 1)
    x = jnp.where(lane < CUST_DIM, cust_ref[...], 0.0)
    h = jnp.dot(x, w1a_ref[...], preferred_element_type=jnp.float32)
    pidx = pidx_ref[0, 0, :]
    oh = (lax.broadcasted_iota(jnp.int32, (_ROWS, PROD_OH), 1)
          == pidx[:, None]).astype(jnp.float32)
    h = h + jnp.dot(oh, p2_ref[...], preferred_element_type=jnp.float32)
    h = h + lax.dot_general(denset_ref[...], w1c_ref[...],
                            (((0,), (0,)), ((), ())),
                            preferred_element_type=jnp.float32)
    h = jnp.maximum(h + b1_ref[...], 0.0)
    h = jnp.dot(h.astype(jnp.bfloat16), w2_ref[...],
                preferred_element_type=jnp.float32) + b2_ref[...]
    h = jnp.maximum(h, 0.0)
    o = jnp.dot(h, w3_ref[...], preferred_element_type=jnp.float32) + b3_ref[...]
    out_ref[...] = 1.0 / (1.0 + jnp.exp(-o))


def _tc_mlp(cust_emb, pidx3, denset, w1a, p2, w1c, b1, w2, b2, w3, b3,
            blk_off, nblk):
    full = lambda shape: pl.BlockSpec(shape, lambda i: (0, 0))
    rows = nblk * _ROWS
    return pl.pallas_call(
        _mlp_body,
        grid=(nblk,),
        in_specs=[
            pl.BlockSpec((_ROWS, TBL_W), lambda i: (i, 0)),
            pl.BlockSpec((1, 1, _ROWS), lambda i: (i + blk_off, 0, 0)),
            pl.BlockSpec((DENSE_DIM, _ROWS), lambda i: (0, i + blk_off)),
            full((TBL_W, H0)),
            full((PROD_OH, H0)),
            full((DENSE_DIM, H0)),
            full((1, H0)),
            full((H0, H1)),
            full((1, H1)),
            full((H1, 1)),
            full((1, 1)),
        ],
        out_specs=pl.BlockSpec((_ROWS, 1), lambda i: (i, 0)),
        out_shape=jax.ShapeDtypeStruct((rows, 1), jnp.float32),
        compiler_params=pltpu.CompilerParams(
            dimension_semantics=("arbitrary",),
        ),
    )(cust_emb, pidx3, denset, w1a, p2, w1c, b1, w2, b2, w3, b3)


_NSPLIT = 1
_BATCH_H = BATCH // _NSPLIT
_NB_H = _BATCH_H // _ROWS


def kernel(core_cust_id_input, prod_code_input, dense_input, W_cust, W_prod,
           W1, b1, W2, b2, W3, b3):
    wc_pad = _tc_transpose_pad(W_cust.T)
    w1a = jnp.pad(W1[:CUST_DIM], ((0, TBL_W - CUST_DIM), (0, 0)))
    p2 = jnp.pad(W_prod @ W1[CUST_DIM:CUST_DIM + PROD_DIM],
                 ((0, PROD_OH - PROD_VOCAB), (0, 0)))
    w1c = W1[CUST_DIM + PROD_DIM:]
    pidx3 = prod_code_input.reshape(_NB, 1, _ROWS)
    denset = dense_input.T
    w2b = W2.astype(jnp.bfloat16)
    gather = _make_sc_gather(_BATCH_H)
    outs = []
    for s in range(_NSPLIT):
        cid = lax.slice_in_dim(core_cust_id_input, s * _BATCH_H,
                               (s + 1) * _BATCH_H)
        ce = gather(cid, wc_pad)
        outs.append(_tc_mlp(ce, pidx3, denset,
                            w1a, p2, w1c, b1.reshape(1, H0),
                            w2b, b2.reshape(1, H1), W3, b3.reshape(1, 1),
                            s * _NB_H, _NB_H))
    return jnp.concatenate(outs, axis=0)
